# Initial kernel scaffold; baseline (speedup 1.0000x reference)
import jax
import jax.numpy as jnp
from jax.experimental import pallas as pl


def kernel(chunk_features, cells, flat_nodes_t, cell_asgn_t, W_phi0, b_phi0, W_phi1, b_phi1, W_rho0, b_rho0, W_rho1, b_rho1, W_rho2, b_rho2):
    raise NotImplementedError("write your pallas kernel here")


# speedup vs baseline: 3.2006x; 3.2006x over previous
import functools

import jax
import jax.numpy as jnp
from jax import lax
from jax.experimental import pallas as pl
from jax.experimental.pallas import tpu as pltpu
from jax.experimental.pallas import tpu_sc as plsc

_N_NODES = 10000
_N_FLAT = 320000
_M = 10000
_D = 128

_NC = 2
_NS = 16
_NW = _NC * _NS
_CHUNK = 128
_G = 2
_NCH = 80
_PAIRS_PAD = _NW * _NCH * _CHUNK
_NCELL_PAD = 10240
_ROWS_PER_SUB = _NCELL_PAD // _NS


def _phi_body(x_ref, w0_ref, b0_ref, w1_ref, b1_ref, o_ref):
    h = jnp.dot(x_ref[...], w0_ref[...], preferred_element_type=jnp.float32)
    h = jnp.maximum(h + b0_ref[...], 0.0)
    h = jnp.dot(h, w1_ref[...], preferred_element_type=jnp.float32)
    o_ref[...] = jnp.maximum(h + b1_ref[...], 0.0)


def _rho_body(p0_ref, p1_ref, w0_ref, b0_ref, w1_ref, b1_ref, w2_ref, b2_ref,
              o_ref):
    s = p0_ref[...] + p1_ref[...]
    r = jnp.dot(s, w0_ref[...], preferred_element_type=jnp.float32)
    r = jnp.maximum(r + b0_ref[...], 0.0)
    r = jnp.dot(r, w1_ref[...], preferred_element_type=jnp.float32)
    r = jnp.maximum(r + b1_ref[...], 0.0)
    r = jnp.dot(r, w2_ref[...], preferred_element_type=jnp.float32)
    o_ref[...] = r + b2_ref[...]


def _sc_body(nodes_r, cells_r, phi_r, zeros_r, out_r,
             nidx, cidx, rows, acc, sem0, sem1):
    c = lax.axis_index("c")
    s = lax.axis_index("s")
    w = c * _NS + s
    pltpu.sync_copy(zeros_r, acc.at[pl.ds(s * _ROWS_PER_SUB, _ROWS_PER_SUB)])
    plsc.subcore_barrier()

    sems = [sem0, sem1]

    def group(g, carry):
        cps = []
        for b in range(_G):
            j = g * _G + b
            pltpu.sync_copy(nodes_r.at[w, j], nidx.at[b])
            pltpu.sync_copy(cells_r.at[w, j], cidx.at[b])
            cp = pltpu.make_async_copy(phi_r.at[nidx.at[b]], rows.at[b],
                                       sems[b])
            cp.start()
            cps.append(cp)
        for b in range(_G):
            cps[b].wait()
            pltpu.sync_copy(rows.at[b], acc.at[cidx.at[b]], add=True)
        return carry

    lax.fori_loop(0, _NCH // _G, group, 0)

    plsc.subcore_barrier()
    sl = pl.ds(s * _ROWS_PER_SUB, _ROWS_PER_SUB)
    pltpu.sync_copy(acc.at[sl], out_r.at[c, sl])


def _segment_scatter_add(nodes, cells_p, phi_all, zeros):
    mesh = plsc.VectorSubcoreMesh(core_axis_name="c", subcore_axis_name="s")
    f = pl.kernel(
        _sc_body,
        out_type=jax.ShapeDtypeStruct((_NC, _NCELL_PAD, _D), jnp.float32),
        mesh=mesh,
        scratch_types=[
            pltpu.VMEM((_G, _CHUNK), jnp.int32),
            pltpu.VMEM((_G, _CHUNK), jnp.int32),
            pltpu.VMEM((_G, _CHUNK, _D), jnp.float32),
            pltpu.VMEM_SHARED((_NCELL_PAD, _D), jnp.float32),
            pltpu.SemaphoreType.DMA,
            pltpu.SemaphoreType.DMA,
        ],
    )
    return f(nodes, cells_p, phi_all, zeros)


def kernel(chunk_features, cells, flat_nodes_t, cell_asgn_t,
           W_phi0, b_phi0, W_phi1, b_phi1,
           W_rho0, b_rho0, W_rho1, b_rho1, W_rho2, b_rho2):
    m = cells.shape[0]

    blk = 2000
    grid = _N_NODES // blk
    rep = lambda i: (0, 0)
    phi_all = pl.pallas_call(
        _phi_body,
        out_shape=jax.ShapeDtypeStruct((_N_NODES, _D), jnp.float32),
        grid=(grid,),
        in_specs=[
            pl.BlockSpec((blk, _D), lambda i: (i, 0)),
            pl.BlockSpec((_D, _D), rep),
            pl.BlockSpec((1, _D), rep),
            pl.BlockSpec((_D, _D), rep),
            pl.BlockSpec((1, _D), rep),
        ],
        out_specs=pl.BlockSpec((blk, _D), lambda i: (i, 0)),
    )(chunk_features, W_phi0, b_phi0.reshape(1, _D),
      W_phi1, b_phi1.reshape(1, _D))

    pad = _PAIRS_PAD - _N_FLAT
    nodes = jnp.pad(flat_nodes_t.astype(jnp.int32), (0, pad))
    nodes = nodes.reshape(_NW, _NCH, _CHUNK)
    dummy = m + (jnp.arange(pad, dtype=jnp.int32) % (_NCELL_PAD - _M))
    cells_p = jnp.concatenate([cell_asgn_t.astype(jnp.int32), dummy])
    cells_p = cells_p.reshape(_NW, _NCH, _CHUNK)
    zeros = jnp.zeros((_ROWS_PER_SUB, _D), jnp.float32)

    partials = _segment_scatter_add(nodes, cells_p, phi_all, zeros)

    rblk = 2048
    rgrid = _NCELL_PAD // rblk
    out = pl.pallas_call(
        _rho_body,
        out_shape=jax.ShapeDtypeStruct((_NCELL_PAD, _D), jnp.float32),
        grid=(rgrid,),
        in_specs=[
            pl.BlockSpec((rblk, _D), lambda i: (i, 0)),
            pl.BlockSpec((rblk, _D), lambda i: (i, 0)),
            pl.BlockSpec((_D, _D), rep),
            pl.BlockSpec((1, _D), rep),
            pl.BlockSpec((_D, _D), rep),
            pl.BlockSpec((1, _D), rep),
            pl.BlockSpec((_D, _D), rep),
            pl.BlockSpec((1, _D), rep),
        ],
        out_specs=pl.BlockSpec((rblk, _D), lambda i: (i, 0)),
    )(partials[0], partials[1],
      W_rho0, b_rho0.reshape(1, _D),
      W_rho1, b_rho1.reshape(1, _D),
      W_rho2, b_rho2.reshape(1, _D))

    return out[:m]


# speedup vs baseline: 3.4548x; 1.0794x over previous
import functools

import jax
import jax.numpy as jnp
from jax import lax
from jax.experimental import pallas as pl
from jax.experimental.pallas import tpu as pltpu
from jax.experimental.pallas import tpu_sc as plsc

_N_NODES = 10000
_N_FLAT = 320000
_M = 10000
_D = 128

_NC = 2
_NS = 16
_NW = _NC * _NS
_CHUNK = 128
_G = 2
_NCH = 80
_NCH_P = _NCH + _G
_PAIRS_PAD = _NW * _NCH * _CHUNK
_NCELL_PAD = 10240
_ROWS_PER_SUB = _NCELL_PAD // _NS


def _phi_body(x_ref, w0_ref, b0_ref, w1_ref, b1_ref, o_ref):
    h = jnp.dot(x_ref[...], w0_ref[...], preferred_element_type=jnp.float32)
    h = jnp.maximum(h + b0_ref[...], 0.0)
    h = jnp.dot(h, w1_ref[...], preferred_element_type=jnp.float32)
    o_ref[...] = jnp.maximum(h + b1_ref[...], 0.0)


def _rho_body(p0_ref, p1_ref, w0_ref, b0_ref, w1_ref, b1_ref, w2_ref, b2_ref,
              o_ref):
    s = p0_ref[...] + p1_ref[...]
    r = jnp.dot(s, w0_ref[...], preferred_element_type=jnp.float32)
    r = jnp.maximum(r + b0_ref[...], 0.0)
    r = jnp.dot(r, w1_ref[...], preferred_element_type=jnp.float32)
    r = jnp.maximum(r + b1_ref[...], 0.0)
    r = jnp.dot(r, w2_ref[...], preferred_element_type=jnp.float32)
    o_ref[...] = r + b2_ref[...]


def _sc_body(pairs_r, phi_r, zeros_r, out_r,
             idx, rows, acc, isem0, isem1, gsem0, gsem1):
    c = lax.axis_index("c")
    s = lax.axis_index("s")
    w = c * _NS + s
    isems = [isem0, isem1]
    gsems = [gsem0, gsem1]

    for b in range(_G):
        pltpu.make_async_copy(pairs_r.at[w, b], idx.at[b], isems[b]).start()
    pltpu.sync_copy(zeros_r, acc.at[pl.ds(s * _ROWS_PER_SUB, _ROWS_PER_SUB)])
    plsc.subcore_barrier()

    def group(g, carry):
        for b in range(_G):
            pltpu.make_async_copy(pairs_r.at[w, 0], idx.at[b],
                                  isems[b]).wait()
            pltpu.make_async_copy(phi_r.at[idx.at[b, 0]], rows.at[b],
                                  gsems[b]).start()
        for b in range(_G):
            pltpu.make_async_copy(phi_r.at[idx.at[b, 0]], rows.at[b],
                                  gsems[b]).wait()
            pltpu.sync_copy(rows.at[b], acc.at[idx.at[b, 1]], add=True)
            pltpu.make_async_copy(pairs_r.at[w, (g + 1) * _G + b], idx.at[b],
                                  isems[b]).start()
        return carry

    lax.fori_loop(0, _NCH // _G, group, 0)

    for b in range(_G):
        pltpu.make_async_copy(pairs_r.at[w, 0], idx.at[b], isems[b]).wait()

    plsc.subcore_barrier()
    sl = pl.ds(s * _ROWS_PER_SUB, _ROWS_PER_SUB)
    pltpu.sync_copy(acc.at[sl], out_r.at[c, sl])


def _segment_scatter_add(pairs, phi_all, zeros):
    mesh = plsc.VectorSubcoreMesh(core_axis_name="c", subcore_axis_name="s")
    f = pl.kernel(
        _sc_body,
        out_type=jax.ShapeDtypeStruct((_NC, _NCELL_PAD, _D), jnp.float32),
        mesh=mesh,
        scratch_types=[
            pltpu.VMEM((_G, 2, _CHUNK), jnp.int32),
            pltpu.VMEM((_G, _CHUNK, _D), jnp.float32),
            pltpu.VMEM_SHARED((_NCELL_PAD, _D), jnp.float32),
            pltpu.SemaphoreType.DMA,
            pltpu.SemaphoreType.DMA,
            pltpu.SemaphoreType.DMA,
            pltpu.SemaphoreType.DMA,
        ],
    )
    return f(pairs, phi_all, zeros)


def kernel(chunk_features, cells, flat_nodes_t, cell_asgn_t,
           W_phi0, b_phi0, W_phi1, b_phi1,
           W_rho0, b_rho0, W_rho1, b_rho1, W_rho2, b_rho2):
    m = cells.shape[0]

    blk = 2000
    grid = _N_NODES // blk
    rep = lambda i: (0, 0)
    phi_all = pl.pallas_call(
        _phi_body,
        out_shape=jax.ShapeDtypeStruct((_N_NODES, _D), jnp.float32),
        grid=(grid,),
        in_specs=[
            pl.BlockSpec((blk, _D), lambda i: (i, 0)),
            pl.BlockSpec((_D, _D), rep),
            pl.BlockSpec((1, _D), rep),
            pl.BlockSpec((_D, _D), rep),
            pl.BlockSpec((1, _D), rep),
        ],
        out_specs=pl.BlockSpec((blk, _D), lambda i: (i, 0)),
    )(chunk_features, W_phi0, b_phi0.reshape(1, _D),
      W_phi1, b_phi1.reshape(1, _D))

    pad = _PAIRS_PAD - _N_FLAT
    nodes = jnp.pad(flat_nodes_t.astype(jnp.int32), (0, pad))
    nodes = nodes.reshape(_NW, _NCH, _CHUNK)
    dummy = m + (jnp.arange(pad, dtype=jnp.int32) % (_NCELL_PAD - _M))
    cells_p = jnp.concatenate([cell_asgn_t.astype(jnp.int32), dummy])
    cells_p = cells_p.reshape(_NW, _NCH, _CHUNK)
    pairs = jnp.stack([nodes, cells_p], axis=2)
    pairs = jnp.pad(pairs, ((0, 0), (0, _NCH_P - _NCH), (0, 0), (0, 0)))
    zeros = jnp.zeros((_ROWS_PER_SUB, _D), jnp.float32)

    partials = _segment_scatter_add(pairs, phi_all, zeros)

    rblk = 2048
    rgrid = _NCELL_PAD // rblk
    out = pl.pallas_call(
        _rho_body,
        out_shape=jax.ShapeDtypeStruct((_NCELL_PAD, _D), jnp.float32),
        grid=(rgrid,),
        in_specs=[
            pl.BlockSpec((rblk, _D), lambda i: (i, 0)),
            pl.BlockSpec((rblk, _D), lambda i: (i, 0)),
            pl.BlockSpec((_D, _D), rep),
            pl.BlockSpec((1, _D), rep),
            pl.BlockSpec((_D, _D), rep),
            pl.BlockSpec((1, _D), rep),
            pl.BlockSpec((_D, _D), rep),
            pl.BlockSpec((1, _D), rep),
        ],
        out_specs=pl.BlockSpec((rblk, _D), lambda i: (i, 0)),
    )(partials[0], partials[1],
      W_rho0, b_rho0.reshape(1, _D),
      W_rho1, b_rho1.reshape(1, _D),
      W_rho2, b_rho2.reshape(1, _D))

    return out[:m]


# speedup vs baseline: 5.1696x; 1.4964x over previous
import functools

import jax
import jax.numpy as jnp
from jax import lax
from jax.experimental import pallas as pl
from jax.experimental.pallas import tpu as pltpu
from jax.experimental.pallas import tpu_sc as plsc

_N_NODES = 10000
_N_FLAT = 320000
_M = 10000
_D = 128

_NC = 2
_NS = 16
_NW = _NC * _NS
_CHUNK = 56
_R = 6
_K = 2 * _R
_NCH0 = 180
_NCH1 = 180
_NCH_MAX = max(_NCH0, _NCH1)
_NCH_P = _NCH_MAX + _R
_PAIRS_PAD = _NS * (_NCH0 + _NCH1) * _CHUNK
_NCELL_PAD = 10112
_ROWS_PER_SUB = _NCELL_PAD // _NS


def _phi_body(x_ref, w0_ref, b0_ref, w1_ref, b1_ref, o_ref):
    h = jnp.dot(x_ref[...], w0_ref[...], preferred_element_type=jnp.float32)
    h = jnp.maximum(h + b0_ref[...], 0.0)
    h = jnp.dot(h, w1_ref[...], preferred_element_type=jnp.float32)
    o_ref[...] = jnp.maximum(h + b1_ref[...], 0.0)


def _rho_body(p0_ref, p1_ref, w0_ref, b0_ref, w1_ref, b1_ref, w2_ref, b2_ref,
              o_ref):
    s = p0_ref[...] + p1_ref[...]
    r = jnp.dot(s, w0_ref[...], preferred_element_type=jnp.float32)
    r = jnp.maximum(r + b0_ref[...], 0.0)
    r = jnp.dot(r, w1_ref[...], preferred_element_type=jnp.float32)
    r = jnp.maximum(r + b1_ref[...], 0.0)
    r = jnp.dot(r, w2_ref[...], preferred_element_type=jnp.float32)
    o_ref[...] = r + b2_ref[...]


def _sc_body(pairs_r, phi_r, zeros_r, out_r,
             idx, rows, acc, isems, gsems, ssems):
    c = lax.axis_index("c")
    s = lax.axis_index("s")
    w = s * _NC + c

    for u in range(_R):
        pltpu.make_async_copy(pairs_r.at[w, u], idx.at[u], isems[u]).start()
    pltpu.sync_copy(zeros_r, acc.at[pl.ds(s * _ROWS_PER_SUB, _ROWS_PER_SUB)])
    plsc.subcore_barrier()

    def supergroup(gg, carry):
        j0 = gg * _K
        for h in range(2):
            for b in range(_R):
                u = h * _R + b
                up = (1 - h) * _R + b
                j = j0 + u
                if h == 0:
                    @pl.when(gg > 0)
                    def _():
                        pltpu.make_async_copy(rows.at[b],
                                              acc.at[idx.at[up, 1]],
                                              ssems[b]).wait()
                else:
                    pltpu.make_async_copy(rows.at[b], acc.at[idx.at[up, 1]],
                                          ssems[b]).wait()
                pltpu.make_async_copy(pairs_r.at[w, j + _R], idx.at[up],
                                      isems[up]).start()
                pltpu.make_async_copy(pairs_r.at[w, 0], idx.at[u],
                                      isems[u]).wait()
                pltpu.make_async_copy(phi_r.at[idx.at[u, 0]], rows.at[b],
                                      gsems[b]).start()
            for b in range(_R):
                u = h * _R + b
                pltpu.make_async_copy(phi_r.at[idx.at[u, 0]], rows.at[b],
                                      gsems[b]).wait()
                pltpu.async_copy(rows.at[b], acc.at[idx.at[u, 1]], ssems[b],
                                 add=True)
        return carry

    n_sg = jnp.where(c == 0, _NCH0 // _K, _NCH1 // _K)
    lax.fori_loop(0, n_sg, supergroup, 0)

    for b in range(_R):
        pltpu.make_async_copy(rows.at[b], acc.at[idx.at[_R + b, 1]],
                              ssems[b]).wait()
        pltpu.make_async_copy(pairs_r.at[w, 0], idx.at[b], isems[b]).wait()

    plsc.subcore_barrier()
    sl = pl.ds(s * _ROWS_PER_SUB, _ROWS_PER_SUB)
    pltpu.sync_copy(acc.at[sl], out_r.at[c, sl])


def _segment_scatter_add(pairs, phi_all, zeros):
    mesh = plsc.VectorSubcoreMesh(core_axis_name="c", subcore_axis_name="s")
    f = pl.kernel(
        _sc_body,
        out_type=jax.ShapeDtypeStruct((_NC, _NCELL_PAD, _D), jnp.float32),
        mesh=mesh,
        scratch_types=[
            pltpu.VMEM((_K, 2, _CHUNK), jnp.int32),
            pltpu.VMEM((_R, _CHUNK, _D), jnp.float32),
            pltpu.VMEM_SHARED((_NCELL_PAD, _D), jnp.float32),
            [pltpu.SemaphoreType.DMA] * _K,
            [pltpu.SemaphoreType.DMA] * _R,
            [pltpu.SemaphoreType.DMA] * _R,
        ],
    )
    return f(pairs, phi_all, zeros)


def kernel(chunk_features, cells, flat_nodes_t, cell_asgn_t,
           W_phi0, b_phi0, W_phi1, b_phi1,
           W_rho0, b_rho0, W_rho1, b_rho1, W_rho2, b_rho2):
    m = cells.shape[0]

    blk = 2000
    grid = _N_NODES // blk
    rep = lambda i: (0, 0)
    phi_all = pl.pallas_call(
        _phi_body,
        out_shape=jax.ShapeDtypeStruct((_N_NODES, _D), jnp.float32),
        grid=(grid,),
        in_specs=[
            pl.BlockSpec((blk, _D), lambda i: (i, 0)),
            pl.BlockSpec((_D, _D), rep),
            pl.BlockSpec((1, _D), rep),
            pl.BlockSpec((_D, _D), rep),
            pl.BlockSpec((1, _D), rep),
        ],
        out_specs=pl.BlockSpec((blk, _D), lambda i: (i, 0)),
    )(chunk_features, W_phi0, b_phi0.reshape(1, _D),
      W_phi1, b_phi1.reshape(1, _D))

    pad = _PAIRS_PAD - _N_FLAT
    nodes_f = jnp.pad(flat_nodes_t.astype(jnp.int32), (0, pad))
    dummy = m + (jnp.arange(pad, dtype=jnp.int32) % (_NCELL_PAD - _M))
    cells_f = jnp.concatenate([cell_asgn_t.astype(jnp.int32), dummy])
    counts = jnp.where(jnp.arange(_NW) % _NC == 0, _NCH0, _NCH1)
    offs = jnp.cumsum(counts) - counts
    tgrid = (offs[:, None] * _CHUNK
             + jnp.arange(_NCH_MAX * _CHUNK, dtype=jnp.int32)[None, :])
    tgrid = jnp.minimum(tgrid, _PAIRS_PAD - 1)
    nodes = nodes_f[tgrid].reshape(_NW, _NCH_MAX, _CHUNK)
    cells_p = cells_f[tgrid].reshape(_NW, _NCH_MAX, _CHUNK)
    pairs = jnp.stack([nodes, cells_p], axis=2)
    pairs = jnp.pad(pairs, ((0, 0), (0, _NCH_P - _NCH_MAX), (0, 0), (0, 0)))
    zeros = jnp.zeros((_ROWS_PER_SUB, _D), jnp.float32)

    partials = _segment_scatter_add(pairs, phi_all, zeros)

    rblk = 2000
    rgrid = _M // rblk
    out = pl.pallas_call(
        _rho_body,
        out_shape=jax.ShapeDtypeStruct((_M, _D), jnp.float32),
        grid=(rgrid,),
        in_specs=[
            pl.BlockSpec((rblk, _D), lambda i: (i, 0)),
            pl.BlockSpec((rblk, _D), lambda i: (i, 0)),
            pl.BlockSpec((_D, _D), rep),
            pl.BlockSpec((1, _D), rep),
            pl.BlockSpec((_D, _D), rep),
            pl.BlockSpec((1, _D), rep),
            pl.BlockSpec((_D, _D), rep),
            pl.BlockSpec((1, _D), rep),
        ],
        out_specs=pl.BlockSpec((rblk, _D), lambda i: (i, 0)),
    )(partials[0], partials[1],
      W_rho0, b_rho0.reshape(1, _D),
      W_rho1, b_rho1.reshape(1, _D),
      W_rho2, b_rho2.reshape(1, _D))

    return out
